# baseline (speedup 1.0000x reference)
import functools

import jax
import jax.numpy as jnp
from jax import lax
from jax.experimental import pallas as pl
from jax.experimental.pallas import tpu as pltpu
from jax.experimental.pallas import tpu_sc as plsc


@functools.lru_cache(maxsize=None)
def _build_lookup(B, NBITS, V, D):
    info = plsc.get_sparse_core_info()
    NC, NS, L = info.num_cores, info.num_subcores, info.num_lanes
    NW = NC * NS
    assert B % (NW * 128) == 0 and D % 8 == 0 and V % 128 == 0
    b_per_w = B // NW
    e_per_w = b_per_w * D
    TCL = b_per_w // 128
    assert TCL == 4
    TBLK = V // 128
    OBLK = B // 128
    mesh = plsc.VectorSubcoreMesh(core_axis_name="c", subcore_axis_name="s")

    @functools.partial(
        pl.kernel,
        mesh=mesh,
        out_type=jax.ShapeDtypeStruct((B * D,), jnp.float32),
        compiler_params=pltpu.CompilerParams(
            use_tc_tiling_on_sc=False, needs_layout_passes=False
        ),
        scratch_types=[
            pltpu.VMEM((NBITS, b_per_w), jnp.int32),
            pltpu.VMEM((b_per_w,), jnp.int32),
            pltpu.VMEM((e_per_w,), jnp.int32),
            pltpu.VMEM((e_per_w,), jnp.float32),
            pltpu.SemaphoreType.DMA,
            pltpu.SemaphoreType.DMA,
            pltpu.SemaphoreType.DMA,
            pltpu.SemaphoreType.DMA,
            pltpu.SemaphoreType.DMA,
        ],
    )
    def lookup(bits_hbm, wflat_hbm, out_hbm, bits_v, p_v, eidx_v, gath_v,
               sem, g0, g1, g2, g3):
        wid = lax.axis_index("s") * NC + lax.axis_index("c")
        base = wid * b_per_w
        gsems = [g0, g1, g2, g3]
        q_elems = e_per_w // TCL

        pltpu.sync_copy(bits_hbm.at[:, pl.ds(base, b_per_w)], bits_v)

        gcps = []
        for tcl in range(TCL):
            @pl.loop(tcl * (128 // L), (tcl + 1) * (128 // L))
            def _convert(i):
                v = bits_v[0, pl.ds(i * L, L)]
                for j in range(1, NBITS):
                    v = v + v + bits_v[j, pl.ds(i * L, L)]
                p_v[pl.ds(i * L, L)] = (v >> 7) * 1024 + (v & 127)

            p_chunks = [
                p_v[pl.ds(tcl * 128 + ch * L, L)] for ch in range(128 // L)
            ]

            @pl.loop(0, D)
            def _build(rc):
                off = (rc >> 3) * (TBLK * 1024) + (rc & 7) * 128
                offv = jnp.full((L,), off, jnp.int32)
                qbase = (tcl * D + rc) * 128
                for ch in range(128 // L):
                    eidx_v[pl.ds(qbase + ch * L, L)] = offv + p_chunks[ch]

            gcps.append(
                pltpu.async_copy(
                    wflat_hbm.at[eidx_v.at[pl.ds(tcl * q_elems, q_elems)]],
                    gath_v.at[pl.ds(tcl * q_elems, q_elems)],
                    gsems[tcl],
                )
            )

        out_cps = []
        for tcl in range(TCL):
            gcps[tcl].wait()
            for tr in range(D // 8):
                dst = (tr * OBLK + wid * TCL + tcl) * 8 * 128
                out_cps.append(
                    pltpu.async_copy(
                        gath_v.at[pl.ds((tcl * D + tr * 8) * 128, 8 * 128)],
                        out_hbm.at[pl.ds(dst, 8 * 128)],
                        sem,
                    )
                )
        for cp in out_cps:
            cp.wait()

    return lookup


def kernel(input_ids, table):
    B, NBITS = input_ids.shape
    V, D = table.shape
    bits_t = input_ids.T
    wflat = (
        table.T.reshape(D // 8, 8, V // 128, 128)
        .transpose(0, 2, 1, 3)
        .reshape(V * D)
    )
    out = _build_lookup(B, NBITS, V, D)(bits_t, wflat)
    out = (
        out.reshape(D // 8, B // 128, 8, 128)
        .transpose(0, 2, 1, 3)
        .reshape(D, B)
        .T
    )
    return out
